# baseline (speedup 1.0000x reference)
import functools
import jax
import jax.numpy as jnp
from jax import lax
from jax.experimental import pallas as pl
from jax.experimental.pallas import tpu as pltpu
from jax.experimental.pallas import tpu_sc as plsc

_EMB = 64
_SCALE = 8.0
_LANES = 16
_NC, _NS = 2, 16
_NW = _NC * _NS
_CHUNK = 128
_NBUF = 8
_DEPTH = 6


@functools.lru_cache(maxsize=None)
def _make_kernel(n_tokens):
    assert n_tokens % (_NW * _CHUNK * _NBUF) == 0
    per_w = n_tokens // _NW
    n_chunks = per_w // _CHUNK
    n_groups = n_chunks // _NBUF
    mesh = plsc.VectorSubcoreMesh(core_axis_name="c", subcore_axis_name="s")

    @functools.partial(
        pl.kernel,
        out_type=jax.ShapeDtypeStruct((_NW, n_chunks, _CHUNK, _EMB), jnp.float32),
        mesh=mesh,
        scratch_types=[
            pltpu.VMEM((n_chunks, _CHUNK), jnp.int32),
            [pltpu.VMEM((_CHUNK, _EMB), jnp.float32) for _ in range(_NBUF)],
            [pltpu.SemaphoreType.DMA for _ in range(_NBUF)],
            [pltpu.SemaphoreType.DMA for _ in range(_NBUF)],
        ],
        compiler_params=pltpu.CompilerParams(use_tc_tiling_on_sc=False),
    )
    def gather_kernel(tokens_hbm, table_hbm, out_hbm, idx_v, rows, gsems, osems):
        wid = lax.axis_index("s") * _NC + lax.axis_index("c")
        pltpu.sync_copy(tokens_hbm.at[wid], idx_v)

        for b in range(_DEPTH):
            pltpu.async_copy(table_hbm.at[idx_v.at[b]], rows[b], gsems[b])

        def process(j, b, bb):
            pltpu.make_async_copy(
                table_hbm.at[idx_v.at[j]], rows[b], gsems[b]
            ).wait()

            def scale_row(r, carry2):
                for c in range(_EMB // _LANES):
                    sl = pl.ds(c * _LANES, _LANES)
                    rows[b][r, sl] = rows[b][r, sl] * _SCALE
                return carry2

            lax.fori_loop(0, _CHUNK, scale_row, 0, unroll=4)
            pltpu.async_copy(rows[b], out_hbm.at[wid, j], osems[b])

            @pl.when(j >= _NBUF - _DEPTH)
            def _():
                pltpu.make_async_copy(
                    rows[bb], out_hbm.at[wid, 0], osems[bb]
                ).wait()

            @pl.when(j + _DEPTH < n_chunks)
            def _():
                pltpu.async_copy(
                    table_hbm.at[idx_v.at[j + _DEPTH]], rows[bb], gsems[bb]
                )

        def group_body(g, carry):
            for b in range(_NBUF):
                process(g * _NBUF + b, b, (b + _DEPTH) % _NBUF)
            return carry

        lax.fori_loop(0, n_groups, group_body, 0)

        for j_tail in range(n_chunks - (_NBUF - _DEPTH), n_chunks):
            b = j_tail % _NBUF
            pltpu.make_async_copy(
                rows[b], out_hbm.at[wid, 0], osems[b]
            ).wait()

    return gather_kernel


@jax.jit
def kernel(tokens, table):
    batch, seq = tokens.shape
    n_tokens = batch * seq
    n_sc = 21 * _NW * _CHUNK * _NBUF
    flat = tokens.reshape(n_tokens)
    tokens_sc = flat[:n_sc].reshape(_NW, n_sc // (_NW * _CHUNK), _CHUNK)
    out_sc = _make_kernel(n_sc)(tokens_sc, table).reshape(n_sc, _EMB)
    out_tc = jnp.take(table, flat[n_sc:], axis=0) * _SCALE
    return jnp.concatenate([out_sc, out_tc], axis=0).reshape(batch, seq, _EMB)


# speedup vs baseline: 2.2151x; 2.2151x over previous
import functools
import jax
import jax.numpy as jnp
from jax import lax
from jax.experimental import pallas as pl
from jax.experimental.pallas import tpu as pltpu
from jax.experimental.pallas import tpu_sc as plsc

_EMB = 64
_SCALE = 8.0
_LANES = 16
_NC, _NS = 2, 16
_NW = _NC * _NS
_CHUNK = 128
_NBUF = 8
_DEPTH = 6


@functools.lru_cache(maxsize=None)
def _make_kernel(n_tokens):
    assert n_tokens % (_NW * _CHUNK * _NBUF) == 0
    per_w = n_tokens // _NW
    n_chunks = per_w // _CHUNK
    n_groups = n_chunks // _NBUF
    mesh = plsc.VectorSubcoreMesh(core_axis_name="c", subcore_axis_name="s")

    @functools.partial(
        pl.kernel,
        out_type=jax.ShapeDtypeStruct((_NW, n_chunks, _CHUNK, _EMB), jnp.float32),
        mesh=mesh,
        scratch_types=[
            pltpu.VMEM((n_chunks, _CHUNK), jnp.int32),
            [pltpu.VMEM((_CHUNK, _EMB), jnp.float32) for _ in range(_NBUF)],
            [pltpu.SemaphoreType.DMA for _ in range(_NBUF)],
            [pltpu.SemaphoreType.DMA for _ in range(_NBUF)],
        ],
        compiler_params=pltpu.CompilerParams(use_tc_tiling_on_sc=False),
    )
    def gather_kernel(tokens_hbm, table_hbm, out_hbm, idx_v, rows, gsems, osems):
        wid = lax.axis_index("s") * _NC + lax.axis_index("c")
        pltpu.sync_copy(tokens_hbm.at[wid], idx_v)

        for b in range(_DEPTH):
            pltpu.async_copy(table_hbm.at[idx_v.at[b]], rows[b], gsems[b])

        def process(j, b, bb):
            pltpu.make_async_copy(
                table_hbm.at[idx_v.at[j]], rows[b], gsems[b]
            ).wait()

            def scale_row(r, carry2):
                for c in range(_EMB // _LANES):
                    sl = pl.ds(c * _LANES, _LANES)
                    rows[b][r, sl] = rows[b][r, sl] * _SCALE
                return carry2

            lax.fori_loop(0, _CHUNK, scale_row, 0, unroll=4)
            pltpu.async_copy(rows[b], out_hbm.at[wid, j], osems[b])

            @pl.when(j >= _NBUF - _DEPTH)
            def _():
                pltpu.make_async_copy(
                    rows[bb], out_hbm.at[wid, 0], osems[bb]
                ).wait()

            @pl.when(j + _DEPTH < n_chunks)
            def _():
                pltpu.async_copy(
                    table_hbm.at[idx_v.at[j + _DEPTH]], rows[bb], gsems[bb]
                )

        def group_body(g, carry):
            for b in range(_NBUF):
                process(g * _NBUF + b, b, (b + _DEPTH) % _NBUF)
            return carry

        lax.fori_loop(0, n_groups, group_body, 0)

        for j_tail in range(n_chunks - (_NBUF - _DEPTH), n_chunks):
            b = j_tail % _NBUF
            pltpu.make_async_copy(
                rows[b], out_hbm.at[wid, 0], osems[b]
            ).wait()

    return gather_kernel


@jax.jit
def kernel(tokens, table):
    batch, seq = tokens.shape
    n_tokens = batch * seq
    tokens_flat = tokens.reshape(_NW, n_tokens // (_NW * _CHUNK), _CHUNK)
    out = _make_kernel(n_tokens)(tokens_flat, table)
    return out.reshape(batch, seq, _EMB)


# speedup vs baseline: 2.2186x; 1.0016x over previous
import functools
import jax
import jax.numpy as jnp
from jax import lax
from jax.experimental import pallas as pl
from jax.experimental.pallas import tpu as pltpu
from jax.experimental.pallas import tpu_sc as plsc

_EMB = 64
_SCALE = 8.0
_LANES = 16
_NC, _NS = 2, 16
_NW = _NC * _NS
_CHUNK = 128
_NBUF = 10
_DEPTH = 8


@functools.lru_cache(maxsize=None)
def _make_kernel(n_tokens):
    assert n_tokens % (_NW * _CHUNK * _NBUF) == 0
    per_w = n_tokens // _NW
    n_chunks = per_w // _CHUNK
    n_groups = n_chunks // _NBUF
    mesh = plsc.VectorSubcoreMesh(core_axis_name="c", subcore_axis_name="s")

    @functools.partial(
        pl.kernel,
        out_type=jax.ShapeDtypeStruct((_NW, n_chunks, _CHUNK, _EMB), jnp.float32),
        mesh=mesh,
        scratch_types=[
            pltpu.VMEM((n_chunks, _CHUNK), jnp.int32),
            [pltpu.VMEM((_CHUNK, _EMB), jnp.float32) for _ in range(_NBUF)],
            [pltpu.SemaphoreType.DMA for _ in range(_NBUF)],
            [pltpu.SemaphoreType.DMA for _ in range(_NBUF)],
        ],
        compiler_params=pltpu.CompilerParams(use_tc_tiling_on_sc=False),
    )
    def gather_kernel(tokens_hbm, table_hbm, out_hbm, idx_v, rows, gsems, osems):
        wid = lax.axis_index("s") * _NC + lax.axis_index("c")
        pltpu.sync_copy(tokens_hbm.at[wid], idx_v)

        for b in range(_DEPTH):
            pltpu.async_copy(table_hbm.at[idx_v.at[b]], rows[b], gsems[b])

        def process(j, b, bb):
            pltpu.make_async_copy(
                table_hbm.at[idx_v.at[j]], rows[b], gsems[b]
            ).wait()

            def scale_row(r, carry2):
                for c in range(_EMB // _LANES):
                    sl = pl.ds(c * _LANES, _LANES)
                    rows[b][r, sl] = rows[b][r, sl] * _SCALE
                return carry2

            lax.fori_loop(0, _CHUNK, scale_row, 0, unroll=4)
            pltpu.async_copy(rows[b], out_hbm.at[wid, j], osems[b])

            @pl.when(j >= _NBUF - _DEPTH)
            def _():
                pltpu.make_async_copy(
                    rows[bb], out_hbm.at[wid, 0], osems[bb]
                ).wait()

            @pl.when(j + _DEPTH < n_chunks)
            def _():
                pltpu.async_copy(
                    table_hbm.at[idx_v.at[j + _DEPTH]], rows[bb], gsems[bb]
                )

        def group_body(g, carry):
            for b in range(_NBUF):
                process(g * _NBUF + b, b, (b + _DEPTH) % _NBUF)
            return carry

        lax.fori_loop(0, n_groups, group_body, 0)

        for j_tail in range(n_chunks - (_NBUF - _DEPTH), n_chunks):
            b = j_tail % _NBUF
            pltpu.make_async_copy(
                rows[b], out_hbm.at[wid, 0], osems[b]
            ).wait()

    return gather_kernel


@jax.jit
def kernel(tokens, table):
    batch, seq = tokens.shape
    n_tokens = batch * seq
    tokens_flat = tokens.reshape(_NW, n_tokens // (_NW * _CHUNK), _CHUNK)
    out = _make_kernel(n_tokens)(tokens_flat, table)
    return out.reshape(batch, seq, _EMB)
